# baseline (speedup 1.0000x reference)
import jax
import jax.numpy as jnp
from jax import lax
from jax.experimental import pallas as pl
from jax.experimental.pallas import tpu as pltpu

_E = 16
_INV_TEMP = 2.0
_STEPS = 49


def _router_body(x_ref, w_ref, b_ref, o_ref, acc_ref):
    i = pl.program_id(0)
    part = jnp.sum(x_ref[...], axis=0)

    @pl.when(i == 0)
    def _init():
        acc_ref[...] = part

    @pl.when(i > 0)
    def _acc():
        acc_ref[...] += part

    @pl.when(i == pl.num_programs(0) - 1)
    def _finish():
        hw = x_ref.shape[0] * pl.num_programs(0)
        pooled = acc_ref[...] * (1.0 / hw)
        logits = lax.dot_general(
            w_ref[...], pooled, (((1,), (1,)), ((), ())),
            preferred_element_type=jnp.float32)
        logits = (logits + b_ref[...]) * _INV_TEMP
        m = jnp.max(logits, axis=0, keepdims=True)
        e = jnp.exp(logits - m)
        o_ref[...] = e / jnp.sum(e, axis=0, keepdims=True)


def kernel(x, W, b):
    B, C = x.shape[0], x.shape[1]
    HW = 1
    for d in x.shape[2:]:
        HW *= d
    xt = jnp.transpose(x, (2, 3, 0, 1)).reshape(HW, B, C)
    b2 = b.reshape(_E, 1)
    s = HW // _STEPS
    out_t = pl.pallas_call(
        _router_body,
        grid=(_STEPS,),
        in_specs=[
            pl.BlockSpec((s, B, C), lambda i: (i, 0, 0)),
            pl.BlockSpec((_E, C), lambda i: (0, 0)),
            pl.BlockSpec((_E, 1), lambda i: (0, 0)),
        ],
        out_specs=pl.BlockSpec((_E, B), lambda i: (0, 0)),
        out_shape=jax.ShapeDtypeStruct((_E, B), jnp.float32),
        scratch_shapes=[pltpu.VMEM((B, C), jnp.float32)],
    )(xt, W, b2)
    return out_t.T


# speedup vs baseline: 1.6044x; 1.6044x over previous
import jax
import jax.numpy as jnp
from jax import lax
from jax.experimental import pallas as pl
from jax.experimental.pallas import tpu as pltpu

_E = 16
_INV_TEMP = 2.0
_STEPS = 16


def _router_body(x_ref, w_ref, b_ref, o_ref, acc_ref):
    i = pl.program_id(0)
    part = jnp.sum(x_ref[...], axis=0)

    @pl.when(i == 0)
    def _init():
        acc_ref[...] = part

    @pl.when(i > 0)
    def _acc():
        acc_ref[...] += part

    @pl.when(i == pl.num_programs(0) - 1)
    def _finish():
        hw = x_ref.shape[0] * pl.num_programs(0)
        pooled = acc_ref[...] * (1.0 / hw)
        logits = lax.dot_general(
            w_ref[...], pooled, (((1,), (1,)), ((), ())),
            preferred_element_type=jnp.float32)
        logits = (logits + b_ref[...]) * _INV_TEMP
        m = jnp.max(logits, axis=0, keepdims=True)
        e = jnp.exp(logits - m)
        o_ref[...] = e / jnp.sum(e, axis=0, keepdims=True)


def kernel(x, W, b):
    B, C = x.shape[0], x.shape[1]
    HW = 1
    for d in x.shape[2:]:
        HW *= d
    xt = jnp.transpose(x, (2, 3, 0, 1)).reshape(HW, B, C)
    b2 = b.reshape(_E, 1)
    s = HW // _STEPS
    out_t = pl.pallas_call(
        _router_body,
        grid=(_STEPS,),
        in_specs=[
            pl.BlockSpec((s, B, C), lambda i: (i, 0, 0)),
            pl.BlockSpec((_E, C), lambda i: (0, 0)),
            pl.BlockSpec((_E, 1), lambda i: (0, 0)),
        ],
        out_specs=pl.BlockSpec((_E, B), lambda i: (0, 0)),
        out_shape=jax.ShapeDtypeStruct((_E, B), jnp.float32),
        scratch_shapes=[pltpu.VMEM((B, C), jnp.float32)],
    )(xt, W, b2)
    return out_t.T
